# Initial kernel scaffold; baseline (speedup 1.0000x reference)
import jax
import jax.numpy as jnp
from jax.experimental import pallas as pl


def kernel(tokens, tables, W_embed_lin, b_embed_lin, W_final, b_final):
    raise NotImplementedError("write your pallas kernel here")


# speedup vs baseline: 8.0942x; 8.0942x over previous
import functools

import jax
import jax.numpy as jnp
from jax import lax
from jax.experimental import pallas as pl
from jax.experimental.pallas import tpu as pltpu
from jax.experimental.pallas import tpu_sc as plsc

N_TABLES = 26
VOCAB = 100000
EMB = 32
HID = 64
NL = 2
B = 16384

R = B * N_TABLES
NC, NS = 2, 16
NW = NC * NS
RPW = R // NW
G = 128
NGPW = RPW // G
SCG = 8
CHUNK = G * SCG
NCH = NGPW // SCG


@functools.partial(
    pl.kernel,
    out_type=jax.ShapeDtypeStruct((R, EMB), jnp.float32),
    mesh=plsc.VectorSubcoreMesh(core_axis_name="c", subcore_axis_name="s"),
    compiler_params=pltpu.CompilerParams(use_tc_tiling_on_sc=False),
    scratch_types=[
        pltpu.VMEM((NGPW, G), jnp.int32),
        pltpu.VMEM((CHUNK, EMB), jnp.float32),
        pltpu.SemaphoreType.DMA,
    ],
)
def _sc_gather(tokens_hbm, table_hbm, out_hbm, idx_v, buf, sem):
    wid = lax.axis_index("s") * NC + lax.axis_index("c")
    base = wid * RPW

    pltpu.sync_copy(tokens_hbm.at[pl.ds(wid * NGPW, NGPW), :], idx_v)

    lanes = lax.iota(jnp.int32, 16)

    def idx_body(j, carry):
        for s in range(G // 16):
            r0 = base + j * G + s * 16
            rows = r0 + lanes
            off = (rows % N_TABLES) * VOCAB
            sl = pl.ds(s * 16, 16)
            idx_v[j, sl] = idx_v[j, sl] + off
        return carry

    lax.fori_loop(0, NGPW, idx_body, 0)

    def chunk_body(c, carry):
        copies = []
        for g in range(SCG):
            copies.append(
                pltpu.async_copy(
                    table_hbm.at[idx_v.at[c * SCG + g]],
                    buf.at[pl.ds(g * G, G), :],
                    sem,
                )
            )
        for cp in copies:
            cp.wait()
        pltpu.sync_copy(buf, out_hbm.at[pl.ds(base + c * CHUNK, CHUNK), :])
        return carry

    lax.fori_loop(0, NCH, chunk_body, 0)


BLK = 2048


def _mm_body(e_ref, m_ref, c_ref, o_ref):
    acc = jnp.dot(e_ref[...], m_ref[...], preferred_element_type=jnp.float32)
    acc = acc + c_ref[...]
    o_ref[0] = acc[:, :HID]
    o_ref[1] = acc[:, HID:]


_mm = pl.pallas_call(
    _mm_body,
    grid=(B // BLK,),
    in_specs=[
        pl.BlockSpec((BLK, N_TABLES * EMB), lambda i: (i, 0)),
        pl.BlockSpec((N_TABLES * EMB, NL * HID), lambda i: (0, 0)),
        pl.BlockSpec((1, NL * HID), lambda i: (0, 0)),
    ],
    out_specs=pl.BlockSpec((NL, BLK, HID), lambda i: (0, i, 0)),
    out_shape=jax.ShapeDtypeStruct((NL, B, HID), jnp.float32),
)


def kernel(tokens, tables, W_embed_lin, b_embed_lin, W_final, b_final):
    tokens_flat = tokens.astype(jnp.int32).reshape(R // G, G)
    table_flat = tables.reshape(N_TABLES * VOCAB, EMB)

    M = jnp.einsum("ln,nhd->ndlh", W_final, W_embed_lin).reshape(
        N_TABLES * EMB, NL * HID
    )
    const = (W_final @ b_embed_lin + b_final[:, None]).reshape(1, NL * HID)

    rows = _sc_gather(tokens_flat, table_flat)
    E = rows.reshape(B, N_TABLES * EMB)
    return _mm(E, M, const)
